# baseline (speedup 1.0000x reference)
import functools

import jax
import jax.numpy as jnp
from jax import lax
from jax.experimental import pallas as pl
from jax.experimental.pallas import tpu as pltpu
from jax.experimental.pallas import tpu_sc as plsc

_INFO = plsc.get_sparse_core_info()
_NC = _INFO.num_cores
_NS = _INFO.num_subcores
_NW = _NC * _NS
_B1 = 256


def _embed_lookup_t(ids_flat_t, table, b, h):
    d = table.shape[1]
    n_units = (b // _B1) * h
    per_w = n_units // _NW
    blk_per_t = b // _B1
    mesh = plsc.VectorSubcoreMesh(core_axis_name="c", subcore_axis_name="s")

    @functools.partial(
        pl.kernel,
        mesh=mesh,
        compiler_params=pltpu.CompilerParams(
            use_tc_tiling_on_sc=False, needs_layout_passes=False
        ),
        out_type=jax.ShapeDtypeStruct((h, d, b), jnp.float32),
        scratch_types=[
            pltpu.VMEM((per_w * _B1,), jnp.int32),
            pltpu.VMEM((_B1, d), jnp.float32),
            pltpu.VMEM((_B1, d), jnp.float32),
            pltpu.VMEM((d, _B1), jnp.float32),
            pltpu.VMEM((d, _B1), jnp.float32),
            pltpu.SemaphoreType.DMA,
            pltpu.SemaphoreType.DMA,
            pltpu.SemaphoreType.DMA,
            pltpu.SemaphoreType.DMA,
        ],
    )
    def k(ids_hbm, table_hbm, out_hbm, idx_v, g_a, g_b, s_a, s_b, ga, gb, sa, sb):
        wid = lax.axis_index("s") * _NC + lax.axis_index("c")
        u0 = wid * per_w
        pltpu.sync_copy(ids_hbm.at[pl.ds(u0 * _B1, per_w * _B1)], idx_v)

        def gather(l, gbuf, sem):
            pltpu.async_copy(table_hbm.at[idx_v.at[pl.ds(l * _B1, _B1)]], gbuf, sem)

        def gather_wait(l, gbuf, sem):
            pltpu.make_async_copy(
                table_hbm.at[idx_v.at[pl.ds(l * _B1, _B1)]], gbuf, sem
            ).wait()

        def out_slice(l):
            u = u0 + l
            t = u // blk_per_t
            b0 = (u % blk_per_t) * _B1
            return out_hbm.at[t, :, pl.ds(b0, _B1)]

        def store(l, sbuf, sem):
            pltpu.async_copy(sbuf, out_slice(l), sem)

        def store_wait(l, sbuf, sem):
            pltpu.make_async_copy(sbuf, out_slice(l), sem).wait()

        lanes = lax.iota(jnp.int32, 16)

        def transpose(gbuf, sbuf):
            def chunk(kk, _):
                rows = lanes + kk * 16
                for j in range(d):
                    col = jnp.full((16,), j, jnp.int32)
                    v = plsc.load_gather(gbuf, [rows, col])
                    sbuf[j, pl.ds(kk * 16, 16)] = v
                return 0

            lax.fori_loop(0, _B1 // 16, chunk, 0)

        gather(0, g_a, ga)
        gather(1, g_b, gb)

        def pair(p, _):
            l0 = 2 * p
            gather_wait(l0, g_a, ga)
            store_wait(l0 - 2, s_a, sa)
            transpose(g_a, s_a)
            store(l0, s_a, sa)
            gather(l0 + 2, g_a, ga)
            gather_wait(l0 + 1, g_b, gb)
            store_wait(l0 - 1, s_b, sb)
            transpose(g_b, s_b)
            store(l0 + 1, s_b, sb)
            gather(l0 + 3, g_b, gb)
            return 0

        gather_wait(0, g_a, ga)
        transpose(g_a, s_a)
        store(0, s_a, sa)
        gather(2, g_a, ga)
        gather_wait(1, g_b, gb)
        transpose(g_b, s_b)
        store(1, s_b, sb)
        gather(3, g_b, gb)

        lax.fori_loop(1, per_w // 2 - 1, pair, 0)

        l0 = per_w - 2
        gather_wait(l0, g_a, ga)
        store_wait(l0 - 2, s_a, sa)
        transpose(g_a, s_a)
        store(l0, s_a, sa)
        gather_wait(l0 + 1, g_b, gb)
        store_wait(l0 - 1, s_b, sb)
        transpose(g_b, s_b)
        store(l0 + 1, s_b, sb)
        store_wait(l0, s_a, sa)
        store_wait(l0 + 1, s_b, sb)

    return k(ids_flat_t, table)


def kernel(input_ids, table):
    b, h = input_ids.shape
    d = table.shape[1]
    ids_t = input_ids.astype(jnp.int32).T.reshape(b * h)
    out3d = _embed_lookup_t(ids_t, table, b, h)
    return out3d.transpose(2, 0, 1)


# speedup vs baseline: 1.1809x; 1.1809x over previous
import functools

import jax
import jax.numpy as jnp
from jax import lax
from jax.experimental import pallas as pl
from jax.experimental.pallas import tpu as pltpu
from jax.experimental.pallas import tpu_sc as plsc

_INFO = plsc.get_sparse_core_info()
_NC = _INFO.num_cores
_NS = _INFO.num_subcores
_NW = _NC * _NS
_B1 = 256


def _embed_lookup_t(ids_flat_t, table, b, h):
    d = table.shape[1]
    n_units = (b // _B1) * h
    per_w = n_units // _NW
    blk_per_t = b // _B1
    mesh = plsc.VectorSubcoreMesh(core_axis_name="c", subcore_axis_name="s")

    @functools.partial(
        pl.kernel,
        mesh=mesh,
        compiler_params=pltpu.CompilerParams(
            use_tc_tiling_on_sc=False, needs_layout_passes=False
        ),
        out_type=jax.ShapeDtypeStruct((h, d, b), jnp.float32),
        scratch_types=[
            pltpu.VMEM((per_w * _B1,), jnp.int32),
            pltpu.VMEM((_B1, d), jnp.float32),
            pltpu.VMEM((_B1, d), jnp.float32),
            pltpu.VMEM((d, _B1), jnp.float32),
            pltpu.VMEM((d, _B1), jnp.float32),
            pltpu.SemaphoreType.DMA,
            pltpu.SemaphoreType.DMA,
            pltpu.SemaphoreType.DMA,
            pltpu.SemaphoreType.DMA,
        ],
    )
    def k(ids_hbm, table_hbm, out_hbm, idx_v, g_a, g_b, s_a, s_b, ga, gb, sa, sb):
        wid = lax.axis_index("s") * _NC + lax.axis_index("c")
        u0 = wid * per_w
        pltpu.sync_copy(ids_hbm.at[pl.ds(u0 * _B1, per_w * _B1)], idx_v)

        def gather(l, gbuf, sem):
            pltpu.async_copy(table_hbm.at[idx_v.at[pl.ds(l * _B1, _B1)]], gbuf, sem)

        def gather_wait(l, gbuf, sem):
            pltpu.make_async_copy(
                table_hbm.at[idx_v.at[pl.ds(l * _B1, _B1)]], gbuf, sem
            ).wait()

        def out_slice(l):
            u = u0 + l
            t = u // blk_per_t
            b0 = (u % blk_per_t) * _B1
            return out_hbm.at[t, :, pl.ds(b0, _B1)]

        def store(l, sbuf, sem):
            pltpu.async_copy(sbuf, out_slice(l), sem)

        def store_wait(l, sbuf, sem):
            pltpu.make_async_copy(sbuf, out_slice(l), sem).wait()

        lanes = lax.iota(jnp.int32, 16)

        def transpose(gbuf, sbuf):
            def jbody(j, _):
                col = jnp.full((16,), 0, jnp.int32) + j
                vs = [
                    plsc.load_gather(gbuf, [lanes + kk * 16, col])
                    for kk in range(_B1 // 16)
                ]
                for kk in range(_B1 // 16):
                    sbuf[j, pl.ds(kk * 16, 16)] = vs[kk]
                return 0

            lax.fori_loop(0, d, jbody, 0)

        gather(0, g_a, ga)
        gather(1, g_b, gb)

        def pair(p, _):
            l0 = 2 * p
            gather_wait(l0, g_a, ga)
            store_wait(l0 - 2, s_a, sa)
            transpose(g_a, s_a)
            store(l0, s_a, sa)
            gather(l0 + 2, g_a, ga)
            gather_wait(l0 + 1, g_b, gb)
            store_wait(l0 - 1, s_b, sb)
            transpose(g_b, s_b)
            store(l0 + 1, s_b, sb)
            gather(l0 + 3, g_b, gb)
            return 0

        gather_wait(0, g_a, ga)
        transpose(g_a, s_a)
        store(0, s_a, sa)
        gather(2, g_a, ga)
        gather_wait(1, g_b, gb)
        transpose(g_b, s_b)
        store(1, s_b, sb)
        gather(3, g_b, gb)

        lax.fori_loop(1, per_w // 2 - 1, pair, 0)

        l0 = per_w - 2
        gather_wait(l0, g_a, ga)
        store_wait(l0 - 2, s_a, sa)
        transpose(g_a, s_a)
        store(l0, s_a, sa)
        gather_wait(l0 + 1, g_b, gb)
        store_wait(l0 - 1, s_b, sb)
        transpose(g_b, s_b)
        store(l0 + 1, s_b, sb)
        store_wait(l0, s_a, sa)
        store_wait(l0 + 1, s_b, sb)

    return k(ids_flat_t, table)


def kernel(input_ids, table):
    b, h = input_ids.shape
    d = table.shape[1]
    ids_t = input_ids.astype(jnp.int32).T.reshape(b * h)
    out3d = _embed_lookup_t(ids_t, table, b, h)
    return out3d.transpose(2, 0, 1)


# speedup vs baseline: 2.4197x; 2.0491x over previous
import functools

import jax
import jax.numpy as jnp
from jax import lax
from jax.experimental import pallas as pl
from jax.experimental.pallas import tpu as pltpu
from jax.experimental.pallas import tpu_sc as plsc

_INFO = plsc.get_sparse_core_info()
_NC = _INFO.num_cores
_NS = _INFO.num_subcores
_NW = _NC * _NS
_B1 = 256


def _embed_lookup_t(ids_flat_t, table, b, h, d):
    dp = table.shape[1]
    n_units = (b // _B1) * h
    per_w = n_units // _NW
    blk_per_t = b // _B1
    mesh = plsc.VectorSubcoreMesh(core_axis_name="c", subcore_axis_name="s")

    @functools.partial(
        pl.kernel,
        mesh=mesh,
        compiler_params=pltpu.CompilerParams(
            use_tc_tiling_on_sc=False, needs_layout_passes=False
        ),
        out_type=jax.ShapeDtypeStruct((h, d, b), jnp.float32),
        scratch_types=[
            pltpu.VMEM((per_w * _B1,), jnp.int32),
            pltpu.VMEM((_B1, dp), jnp.float32),
            pltpu.VMEM((_B1, dp), jnp.float32),
            pltpu.VMEM((d, _B1), jnp.float32),
            pltpu.VMEM((d, _B1), jnp.float32),
            pltpu.SemaphoreType.DMA,
            pltpu.SemaphoreType.DMA,
            pltpu.SemaphoreType.DMA,
            pltpu.SemaphoreType.DMA,
        ],
    )
    def k(ids_hbm, table_hbm, out_hbm, idx_v, g_a, g_b, s_a, s_b, ga, gb, sa, sb):
        wid = lax.axis_index("s") * _NC + lax.axis_index("c")
        u0 = wid * per_w
        pltpu.sync_copy(ids_hbm.at[pl.ds(u0 * _B1, per_w * _B1)], idx_v)

        def gather(l, gbuf, sem):
            pltpu.async_copy(table_hbm.at[idx_v.at[pl.ds(l * _B1, _B1)]], gbuf, sem)

        def gather_wait(l, gbuf, sem):
            pltpu.make_async_copy(
                table_hbm.at[idx_v.at[pl.ds(l * _B1, _B1)]], gbuf, sem
            ).wait()

        def out_slice(l):
            u = u0 + l
            t = u // blk_per_t
            b0 = (u % blk_per_t) * _B1
            return out_hbm.at[t, :, pl.ds(b0, _B1)]

        def store(l, sbuf, sem):
            pltpu.async_copy(sbuf, out_slice(l), sem)

        def store_wait(l, sbuf, sem):
            pltpu.make_async_copy(sbuf, out_slice(l), sem).wait()

        lanes = lax.iota(jnp.int32, 16)

        def transpose(gbuf, sbuf):
            def jbody(j, _):
                col = jnp.full((16,), 0, jnp.int32) + j
                vs = [
                    plsc.load_gather(gbuf, [lanes + kk * 16, col])
                    for kk in range(_B1 // 16)
                ]
                for kk in range(_B1 // 16):
                    sbuf[j, pl.ds(kk * 16, 16)] = vs[kk]
                return 0

            lax.fori_loop(0, d, jbody, 0)

        gather(0, g_a, ga)
        gather(1, g_b, gb)

        def pair(p, _):
            l0 = 2 * p
            gather_wait(l0, g_a, ga)
            store_wait(l0 - 2, s_a, sa)
            transpose(g_a, s_a)
            store(l0, s_a, sa)
            gather(l0 + 2, g_a, ga)
            gather_wait(l0 + 1, g_b, gb)
            store_wait(l0 - 1, s_b, sb)
            transpose(g_b, s_b)
            store(l0 + 1, s_b, sb)
            gather(l0 + 3, g_b, gb)
            return 0

        gather_wait(0, g_a, ga)
        transpose(g_a, s_a)
        store(0, s_a, sa)
        gather(2, g_a, ga)
        gather_wait(1, g_b, gb)
        transpose(g_b, s_b)
        store(1, s_b, sb)
        gather(3, g_b, gb)

        lax.fori_loop(1, per_w // 2 - 1, pair, 0)

        l0 = per_w - 2
        gather_wait(l0, g_a, ga)
        store_wait(l0 - 2, s_a, sa)
        transpose(g_a, s_a)
        store(l0, s_a, sa)
        gather_wait(l0 + 1, g_b, gb)
        store_wait(l0 - 1, s_b, sb)
        transpose(g_b, s_b)
        store(l0 + 1, s_b, sb)
        store_wait(l0, s_a, sa)
        store_wait(l0 + 1, s_b, sb)

    return k(ids_flat_t, table)


def kernel(input_ids, table):
    b, h = input_ids.shape
    d = table.shape[1]
    ids_t = input_ids.astype(jnp.int32).T.reshape(b * h)
    table_p = jnp.pad(table, ((0, 0), (0, 1)))
    out3d = _embed_lookup_t(ids_t, table_p, b, h, d)
    return out3d.transpose(2, 0, 1)
